# Initial kernel scaffold; baseline (speedup 1.0000x reference)
import jax
import jax.numpy as jnp
from jax.experimental import pallas as pl


def kernel(xyz, points, W_kernel, gamma_k, beta_k, W_agg, gamma_a, beta_a, W_lin, b_lin):
    raise NotImplementedError("write your pallas kernel here")


# speedup vs baseline: 2.3521x; 2.3521x over previous
import jax
import jax.numpy as jnp
from jax import lax
from jax.experimental import pallas as pl
from jax.experimental.pallas import tpu as pltpu

LEAKY_RATE = 0.1
EPS = 1e-5
NPOINT = 2048
NSAMPLE = 16


def _leaky(x):
    return jnp.where(x >= 0, x, LEAKY_RATE * x)


def _fps_body(xyz_ref, idx_ref, newxyz_ref):
    B, _, N = xyz_ref.shape
    S = idx_ref.shape[1]
    x = xyz_ref[...]
    iota_n = lax.broadcasted_iota(jnp.int32, (B, N), 1)
    iota_s = lax.broadcasted_iota(jnp.int32, (B, S), 1)

    def extract(last):
        onehot = (iota_n == last).astype(jnp.float32)
        return jnp.sum(x * onehot[:, None, :], axis=2)

    def body(i, carry):
        dists, last, idxs, coords = carry
        pt = extract(last)
        coords = jnp.where(iota_s[:, None, :] == (i - 1), pt[:, :, None], coords)
        d = jnp.sum((x - pt[:, :, None]) ** 2, axis=1)
        dists = jnp.minimum(dists, d)
        maxv = jnp.max(dists, axis=1, keepdims=True)
        nxt = jnp.min(jnp.where(dists == maxv, iota_n, N),
                      axis=1, keepdims=True).astype(jnp.int32)
        idxs = jnp.where(iota_s == i, nxt, idxs)
        return dists, nxt, idxs, coords

    dists0 = jnp.full((B, N), 1e10, dtype=jnp.float32)
    last0 = jnp.zeros((B, 1), jnp.int32)
    idxs0 = jnp.zeros((B, S), jnp.int32)
    coords0 = jnp.zeros((B, 3, S), jnp.float32)
    dists, last, idxs, coords = lax.fori_loop(
        1, S, body, (dists0, last0, idxs0, coords0))
    pt = extract(last)
    coords = jnp.where(iota_s[:, None, :] == (S - 1), pt[:, :, None], coords)
    idx_ref[...] = idxs
    newxyz_ref[...] = coords


def _fps(xyz):
    B, _, N = xyz.shape
    return pl.pallas_call(
        _fps_body,
        out_shape=(jax.ShapeDtypeStruct((B, NPOINT), jnp.int32),
                   jax.ShapeDtypeStruct((B, 3, NPOINT), jnp.float32)),
    )(xyz)


def kernel(xyz, points, W_kernel, gamma_k, beta_k, W_agg, gamma_a, beta_a,
           W_lin, b_lin):
    B = xyz.shape[0]
    fps_idx, new_xyz_cn = _fps(xyz)
    new_xyz = new_xyz_cn.transpose(0, 2, 1)

    xyz_t = xyz.transpose(0, 2, 1)
    points_t = points.transpose(0, 2, 1)

    dist = -2.0 * jnp.matmul(new_xyz, xyz_t.transpose(0, 2, 1))
    dist = dist + jnp.sum(new_xyz ** 2, axis=-1)[:, :, None]
    dist = dist + jnp.sum(xyz_t ** 2, axis=-1)[:, None, :]
    _, idx = jax.lax.top_k(-dist, NSAMPLE)

    bidx = jnp.arange(B)[:, None, None]
    grouped_xyz = xyz_t[bidx, idx]
    grouped_xyz_norm = grouped_xyz - new_xyz[:, :, None, :]
    grouped_points = points_t[bidx, idx]
    new_points = jnp.concatenate([grouped_xyz_norm, grouped_points], axis=-1)

    kern = jnp.einsum('bskc,oc->bsko', new_points, W_kernel)
    m = jnp.mean(kern, axis=(0, 1, 2), keepdims=True)
    v = jnp.mean((kern - m) ** 2, axis=(0, 1, 2), keepdims=True)
    kern = (kern - m) / jnp.sqrt(v + EPS) * gamma_k + beta_k
    kern = _leaky(kern)
    kern = kern.transpose(0, 1, 3, 2)
    aggregation = jnp.matmul(kern, new_points)
    agg = jnp.einsum('bsoc,c->bso', aggregation, W_agg[0])
    m2 = jnp.mean(agg)
    v2 = jnp.mean((agg - m2) ** 2)
    agg = (agg - m2) / jnp.sqrt(v2 + EPS) * gamma_a[0] + beta_a[0]
    agg = _leaky(agg)
    new_feat = agg @ W_lin.T + b_lin
    new_feat = new_feat.transpose(0, 2, 1)
    new_feat = _leaky(new_feat)
    return (new_xyz_cn, new_feat, fps_idx)


# speedup vs baseline: 7.1788x; 3.0521x over previous
import jax
import jax.numpy as jnp
from jax import lax
from jax.experimental import pallas as pl
from jax.experimental.pallas import tpu as pltpu

LEAKY_RATE = 0.1
EPS = 1e-5
NPOINT = 2048
NSAMPLE = 16


def _leaky(x):
    return jnp.where(x >= 0, x, LEAKY_RATE * x)


def _fps_body(xyz_ref, idx_ref, newxyz_ref):
    B, _, N = xyz_ref.shape
    S = idx_ref.shape[1]
    x = xyz_ref[...]
    iota_n = lax.broadcasted_iota(jnp.int32, (B, N), 1)
    iota_s = lax.broadcasted_iota(jnp.int32, (B, S), 1)

    def extract(last):
        onehot = (iota_n == last).astype(jnp.float32)
        return jnp.sum(x * onehot[:, None, :], axis=2)

    def body(i, carry):
        dists, last, idxs, coords = carry
        pt = extract(last)
        coords = jnp.where(iota_s[:, None, :] == (i - 1), pt[:, :, None], coords)
        d = jnp.sum((x - pt[:, :, None]) ** 2, axis=1)
        dists = jnp.minimum(dists, d)
        maxv = jnp.max(dists, axis=1, keepdims=True)
        nxt = jnp.min(jnp.where(dists == maxv, iota_n, N),
                      axis=1, keepdims=True).astype(jnp.int32)
        idxs = jnp.where(iota_s == i, nxt, idxs)
        return dists, nxt, idxs, coords

    dists0 = jnp.full((B, N), 1e10, dtype=jnp.float32)
    last0 = jnp.zeros((B, 1), jnp.int32)
    idxs0 = jnp.zeros((B, S), jnp.int32)
    coords0 = jnp.zeros((B, 3, S), jnp.float32)
    dists, last, idxs, coords = lax.fori_loop(
        1, S, body, (dists0, last0, idxs0, coords0))
    pt = extract(last)
    coords = jnp.where(iota_s[:, None, :] == (S - 1), pt[:, :, None], coords)
    idx_ref[...] = idxs
    newxyz_ref[...] = coords


def _fps(xyz):
    B, _, N = xyz.shape
    return pl.pallas_call(
        _fps_body,
        out_shape=(jax.ShapeDtypeStruct((B, NPOINT), jnp.int32),
                   jax.ShapeDtypeStruct((B, 3, NPOINT), jnp.float32)),
    )(xyz)


_TQ = 256


def _knn_body(newxyz_ref, xyz_ref, idx_ref):
    _, _, N = xyz_ref.shape
    q = newxyz_ref[0]
    x = xyz_ref[0]
    qt = q.T
    inner = jnp.dot(qt, x, preferred_element_type=jnp.float32)
    qsq = jnp.sum(qt * qt, axis=1, keepdims=True)
    xsq = jnp.sum(x * x, axis=0, keepdims=True)
    d = (-2.0 * inner + qsq) + xsq
    iota = lax.broadcasted_iota(jnp.int32, (_TQ, N), 1)
    cols = []
    for _ in range(NSAMPLE):
        amin = jnp.argmin(d, axis=1).astype(jnp.int32)[:, None]
        cols.append(amin)
        d = jnp.where(iota == amin, jnp.inf, d)
    idx_ref[0] = jnp.concatenate(cols, axis=1)


def _knn(new_xyz_cn, xyz):
    B, _, N = xyz.shape
    grid = (B, NPOINT // _TQ)
    return pl.pallas_call(
        _knn_body,
        grid=grid,
        in_specs=[
            pl.BlockSpec((1, 3, _TQ), lambda b, t: (b, 0, t)),
            pl.BlockSpec((1, 3, N), lambda b, t: (b, 0, 0)),
        ],
        out_specs=pl.BlockSpec((1, _TQ, NSAMPLE), lambda b, t: (b, t, 0)),
        out_shape=jax.ShapeDtypeStruct((B, NPOINT, NSAMPLE), jnp.int32),
    )(new_xyz_cn, xyz)


def kernel(xyz, points, W_kernel, gamma_k, beta_k, W_agg, gamma_a, beta_a,
           W_lin, b_lin):
    B = xyz.shape[0]
    fps_idx, new_xyz_cn = _fps(xyz)
    new_xyz = new_xyz_cn.transpose(0, 2, 1)

    xyz_t = xyz.transpose(0, 2, 1)
    points_t = points.transpose(0, 2, 1)

    idx = _knn(new_xyz_cn, xyz)

    bidx = jnp.arange(B)[:, None, None]
    grouped_xyz = xyz_t[bidx, idx]
    grouped_xyz_norm = grouped_xyz - new_xyz[:, :, None, :]
    grouped_points = points_t[bidx, idx]
    new_points = jnp.concatenate([grouped_xyz_norm, grouped_points], axis=-1)

    kern = jnp.einsum('bskc,oc->bsko', new_points, W_kernel)
    m = jnp.mean(kern, axis=(0, 1, 2), keepdims=True)
    v = jnp.mean((kern - m) ** 2, axis=(0, 1, 2), keepdims=True)
    kern = (kern - m) / jnp.sqrt(v + EPS) * gamma_k + beta_k
    kern = _leaky(kern)
    kern = kern.transpose(0, 1, 3, 2)
    aggregation = jnp.matmul(kern, new_points)
    agg = jnp.einsum('bsoc,c->bso', aggregation, W_agg[0])
    m2 = jnp.mean(agg)
    v2 = jnp.mean((agg - m2) ** 2)
    agg = (agg - m2) / jnp.sqrt(v2 + EPS) * gamma_a[0] + beta_a[0]
    agg = _leaky(agg)
    new_feat = agg @ W_lin.T + b_lin
    new_feat = new_feat.transpose(0, 2, 1)
    new_feat = _leaky(new_feat)
    return (new_xyz_cn, new_feat, fps_idx)


# speedup vs baseline: 13.0005x; 1.8109x over previous
import functools

import jax
import jax.numpy as jnp
from jax import lax
from jax.experimental import pallas as pl
from jax.experimental.pallas import tpu as pltpu
from jax.experimental.pallas import tpu_sc as plsc

LEAKY_RATE = 0.1
EPS = 1e-5
NPOINT = 2048
NSAMPLE = 16


def _leaky(x):
    return jnp.where(x >= 0, x, LEAKY_RATE * x)


def _fps_body(xyz_ref, idx_ref, newxyz_ref):
    B, _, N = xyz_ref.shape
    S = idx_ref.shape[1]
    x = xyz_ref[...]
    iota_n = lax.broadcasted_iota(jnp.int32, (B, N), 1)
    iota_s = lax.broadcasted_iota(jnp.int32, (B, S), 1)

    def extract(last):
        onehot = (iota_n == last).astype(jnp.float32)
        return jnp.sum(x * onehot[:, None, :], axis=2)

    def body(i, carry):
        dists, last, idxs, coords = carry
        pt = extract(last)
        coords = jnp.where(iota_s[:, None, :] == (i - 1), pt[:, :, None], coords)
        d = jnp.sum((x - pt[:, :, None]) ** 2, axis=1)
        dists = jnp.minimum(dists, d)
        maxv = jnp.max(dists, axis=1, keepdims=True)
        nxt = jnp.min(jnp.where(dists == maxv, iota_n, N),
                      axis=1, keepdims=True).astype(jnp.int32)
        idxs = jnp.where(iota_s == i, nxt, idxs)
        return dists, nxt, idxs, coords

    dists0 = jnp.full((B, N), 1e10, dtype=jnp.float32)
    last0 = jnp.zeros((B, 1), jnp.int32)
    idxs0 = jnp.zeros((B, S), jnp.int32)
    coords0 = jnp.zeros((B, 3, S), jnp.float32)
    dists, last, idxs, coords = lax.fori_loop(
        1, S, body, (dists0, last0, idxs0, coords0))
    pt = extract(last)
    coords = jnp.where(iota_s[:, None, :] == (S - 1), pt[:, :, None], coords)
    idx_ref[...] = idxs
    newxyz_ref[...] = coords


def _fps(xyz):
    B, _, N = xyz.shape
    return pl.pallas_call(
        _fps_body,
        out_shape=(jax.ShapeDtypeStruct((B, NPOINT), jnp.int32),
                   jax.ShapeDtypeStruct((B, 3, NPOINT), jnp.float32)),
    )(xyz)


_TQ = 256


def _knn_body(newxyz_ref, xyz_ref, idx_ref):
    _, _, N = xyz_ref.shape
    q = newxyz_ref[0]
    x = xyz_ref[0]
    qt = q.T
    inner = jnp.dot(qt, x, preferred_element_type=jnp.float32)
    qsq = jnp.sum(qt * qt, axis=1, keepdims=True)
    xsq = jnp.sum(x * x, axis=0, keepdims=True)
    d = (-2.0 * inner + qsq) + xsq
    iota = lax.broadcasted_iota(jnp.int32, (_TQ, N), 1)
    cols = []
    for _ in range(NSAMPLE):
        amin = jnp.argmin(d, axis=1).astype(jnp.int32)[:, None]
        cols.append(amin)
        d = jnp.where(iota == amin, jnp.inf, d)
    idx_ref[0] = jnp.concatenate(cols, axis=1)


def _knn(new_xyz_cn, xyz):
    B, _, N = xyz.shape
    grid = (B, NPOINT // _TQ)
    return pl.pallas_call(
        _knn_body,
        grid=grid,
        in_specs=[
            pl.BlockSpec((1, 3, _TQ), lambda b, t: (b, 0, t)),
            pl.BlockSpec((1, 3, N), lambda b, t: (b, 0, 0)),
        ],
        out_specs=pl.BlockSpec((1, _TQ, NSAMPLE), lambda b, t: (b, t, 0)),
        out_shape=jax.ShapeDtypeStruct((B, NPOINT, NSAMPLE), jnp.int32),
    )(new_xyz_cn, xyz)


_GROWS = 512
_RPW = _GROWS // 32


def _sc_gather(idx2d, feats):
    C = feats.shape[1]
    mesh = plsc.VectorSubcoreMesh(core_axis_name="c", subcore_axis_name="s")

    @functools.partial(
        pl.kernel,
        mesh=mesh,
        compiler_params=pltpu.CompilerParams(use_tc_tiling_on_sc=False),
        out_type=jax.ShapeDtypeStruct((_GROWS, 128, C), jnp.float32),
        scratch_types=[
            pltpu.VMEM((_RPW, 128), jnp.int32),
            pltpu.VMEM((_RPW, 128, C), jnp.float32),
            pltpu.SemaphoreType.DMA,
        ],
    )
    def gk(idx_hbm, feats_hbm, out_hbm, idx_v, rows_v, sem):
        wid = lax.axis_index("s") * 2 + lax.axis_index("c")
        base = wid * _RPW
        pltpu.sync_copy(idx_hbm.at[pl.ds(base, _RPW)], idx_v)
        copies = [
            pltpu.async_copy(feats_hbm.at[idx_v.at[j]], rows_v.at[j], sem)
            for j in range(_RPW)
        ]
        for cp in copies:
            cp.wait()
        pltpu.sync_copy(rows_v, out_hbm.at[pl.ds(base, _RPW)])

    return gk(idx2d, feats)


_TS1 = 512


def _d1_body(nbr_ref, ctr_ref, g_ref, s_ref):
    t = pl.program_id(0)
    x = nbr_ref[...]
    c = ctr_ref[...]
    cb = jnp.broadcast_to(c[:, None, :], (_TS1, NSAMPLE, 16)).reshape(
        _TS1 * NSAMPLE, 16)
    xn = x - cb
    xb = xn.astype(jnp.bfloat16).astype(jnp.float32)
    g = lax.dot_general(xb, xb, (((0,), (0,)), ((), ())),
                        precision=lax.Precision.HIGHEST,
                        preferred_element_type=jnp.float32)
    s = jnp.broadcast_to(jnp.sum(xb, axis=0, keepdims=True), (8, 16))

    @pl.when(t == 0)
    def _():
        g_ref[...] = g
        s_ref[...] = s

    @pl.when(t > 0)
    def _():
        g_ref[...] = g_ref[...] + g
        s_ref[...] = s_ref[...] + s


def _d2_body(nbr_ref, ctr_ref, wk_ref, par_ref, agg_ref, st_ref):
    x = nbr_ref[...]
    c = ctr_ref[...]
    cb = jnp.broadcast_to(c[:, None, :], (_TS1, NSAMPLE, 16)).reshape(
        _TS1 * NSAMPLE, 16)
    xn = x - cb
    xb = xn.astype(jnp.bfloat16).astype(jnp.float32)
    wkb = wk_ref[...].astype(jnp.bfloat16).astype(jnp.float32)
    kern = lax.dot_general(xb, wkb, (((1,), (1,)), ((), ())),
                           precision=lax.Precision.HIGHEST,
                           preferred_element_type=jnp.float32)
    kern = kern * par_ref[0:1, :] + par_ref[1:2, :]
    kern = _leaky(kern).astype(jnp.bfloat16).astype(jnp.float32)
    y = jnp.sum(xb * par_ref[2:3, :], axis=1, keepdims=True)
    agg = jnp.sum((kern * y).reshape(_TS1, NSAMPLE, 16), axis=1)
    agg_ref[...] = agg
    s1 = jnp.sum(agg)
    s2 = jnp.sum(agg * agg)
    row = lax.broadcasted_iota(jnp.int32, (1, 8, 128), 1)
    st_ref[...] = jnp.where(row == 0, s1, s2)


def _d3_body(agg_ref, wl_ref, bl_ref, ab_ref, out_ref):
    a = ab_ref[0:1, 0:1]
    cshift = ab_ref[1:2, 0:1]
    y = _leaky(agg_ref[...] * a + cshift)
    yb = y.astype(jnp.bfloat16).astype(jnp.float32)
    wlb = wl_ref[...].astype(jnp.bfloat16).astype(jnp.float32)
    z = lax.dot_general(yb, wlb, (((1,), (1,)), ((), ())),
                        precision=lax.Precision.HIGHEST,
                        preferred_element_type=jnp.float32) + bl_ref[...]
    z = _leaky(z)
    out_ref[0] = z.T


def kernel(xyz, points, W_kernel, gamma_k, beta_k, W_agg, gamma_a, beta_a,
           W_lin, b_lin):
    B, _, N = xyz.shape
    S, K = NPOINT, NSAMPLE
    fps_idx, new_xyz_cn = _fps(xyz)
    idx = _knn(new_xyz_cn, xyz)

    feats = jnp.concatenate(
        [xyz.transpose(0, 2, 1), points.transpose(0, 2, 1)], axis=-1
    ).reshape(B * N, 16)
    idxf = (idx + (jnp.arange(B, dtype=jnp.int32) * N)[:, None, None]
            ).reshape(_GROWS, 128)
    nbr = _sc_gather(idxf, feats).reshape(B * S * K, 16)

    ctr16 = jnp.concatenate(
        [new_xyz_cn.transpose(0, 2, 1).reshape(B * S, 3),
         jnp.zeros((B * S, 13), jnp.float32)], axis=1)

    T1 = (B * S) // _TS1
    g, s8 = pl.pallas_call(
        _d1_body,
        grid=(T1,),
        in_specs=[
            pl.BlockSpec((_TS1 * K, 16), lambda t: (t, 0)),
            pl.BlockSpec((_TS1, 16), lambda t: (t, 0)),
        ],
        out_specs=(pl.BlockSpec((16, 16), lambda t: (0, 0)),
                   pl.BlockSpec((8, 16), lambda t: (0, 0))),
        out_shape=(jax.ShapeDtypeStruct((16, 16), jnp.float32),
                   jax.ShapeDtypeStruct((8, 16), jnp.float32)),
    )(nbr, ctr16)

    M = B * S * K
    wkb = W_kernel.astype(jnp.bfloat16).astype(jnp.float32)
    xbar = s8[0] / M
    mean = wkb @ xbar
    e2 = jnp.sum((wkb @ (g / M)) * wkb, axis=1)
    var = e2 - mean ** 2
    scale = gamma_k / jnp.sqrt(var + EPS)
    shift = beta_k - mean * scale
    params = jnp.zeros((8, 16), jnp.float32)
    waggb = W_agg[0].astype(jnp.bfloat16).astype(jnp.float32)
    params = params.at[0].set(scale).at[1].set(shift).at[2].set(waggb)

    agg, st = pl.pallas_call(
        _d2_body,
        grid=(T1,),
        in_specs=[
            pl.BlockSpec((_TS1 * K, 16), lambda t: (t, 0)),
            pl.BlockSpec((_TS1, 16), lambda t: (t, 0)),
            pl.BlockSpec((16, 16), lambda t: (0, 0)),
            pl.BlockSpec((8, 16), lambda t: (0, 0)),
        ],
        out_specs=(pl.BlockSpec((_TS1, 16), lambda t: (t, 0)),
                   pl.BlockSpec((1, 8, 128), lambda t: (t, 0, 0))),
        out_shape=(jax.ShapeDtypeStruct((B * S, 16), jnp.float32),
                   jax.ShapeDtypeStruct((T1, 8, 128), jnp.float32)),
    )(nbr, ctr16, W_kernel, params)

    M2 = B * S * 16
    s1 = jnp.sum(st[:, 0, 0])
    s2 = jnp.sum(st[:, 1, 0])
    m2 = s1 / M2
    v2 = s2 / M2 - m2 ** 2
    a = gamma_a[0] / jnp.sqrt(v2 + EPS)
    cshift = beta_a[0] - m2 * a
    ab = jnp.zeros((8, 128), jnp.float32).at[0, 0].set(a).at[1, 0].set(cshift)

    TS3 = 1024
    new_feat = pl.pallas_call(
        _d3_body,
        grid=(B, S // TS3),
        in_specs=[
            pl.BlockSpec((TS3, 16), lambda b, t: (b * (S // TS3) + t, 0)),
            pl.BlockSpec((16, 16), lambda b, t: (0, 0)),
            pl.BlockSpec((1, 16), lambda b, t: (0, 0)),
            pl.BlockSpec((8, 128), lambda b, t: (0, 0)),
        ],
        out_specs=pl.BlockSpec((1, 16, TS3), lambda b, t: (b, 0, t)),
        out_shape=jax.ShapeDtypeStruct((B, 16, S), jnp.float32),
    )(agg, W_lin, b_lin.reshape(1, 16), ab)

    return (new_xyz_cn, new_feat, fps_idx)


# speedup vs baseline: 18.4570x; 1.4197x over previous
import functools

import jax
import jax.numpy as jnp
from jax import lax
from jax.experimental import pallas as pl
from jax.experimental.pallas import tpu as pltpu
from jax.experimental.pallas import tpu_sc as plsc

LEAKY_RATE = 0.1
EPS = 1e-5
NPOINT = 2048
NSAMPLE = 16


def _leaky(x):
    return jnp.where(x >= 0, x, LEAKY_RATE * x)


_NLANE = 1024


def _fps_body(xyz_ref, idx_ref, newxyz_ref):
    B, _, NSUB, NL = xyz_ref.shape
    N = NSUB * NL
    S = idx_ref.shape[1]
    SSUB = S // NL
    x = xyz_ref[...]
    iota_n = (lax.broadcasted_iota(jnp.int32, (B, NSUB, NL), 1) * NL
              + lax.broadcasted_iota(jnp.int32, (B, NSUB, NL), 2))
    iota_s = (lax.broadcasted_iota(jnp.int32, (B, SSUB, NL), 1) * NL
              + lax.broadcasted_iota(jnp.int32, (B, SSUB, NL), 2))

    def extract(last):
        onehot = (iota_n == last).astype(jnp.float32)
        return jnp.sum(jnp.sum(x * onehot[:, None], axis=3), axis=2)

    def body(i, carry):
        dists, last, idxs, coords = carry
        pt = extract(last)
        coords = jnp.where(iota_s[:, None] == (i - 1),
                           pt[:, :, None, None], coords)
        d = jnp.sum((x - pt[:, :, None, None]) ** 2, axis=1)
        dists = jnp.minimum(dists, d)
        maxv = jnp.max(jnp.max(dists, axis=2, keepdims=True),
                       axis=1, keepdims=True)
        cand = jnp.where(dists == maxv, iota_n, N)
        nxt = jnp.min(jnp.min(cand, axis=2, keepdims=True),
                      axis=1, keepdims=True).astype(jnp.int32)
        idxs = jnp.where(iota_s == i, nxt, idxs)
        return dists, nxt, idxs, coords

    dists0 = jnp.full((B, NSUB, NL), 1e10, dtype=jnp.float32)
    last0 = jnp.zeros((B, 1, 1), jnp.int32)
    idxs0 = jnp.zeros((B, SSUB, NL), jnp.int32)
    coords0 = jnp.zeros((B, 3, SSUB, NL), jnp.float32)
    dists, last, idxs, coords = lax.fori_loop(
        1, S, body, (dists0, last0, idxs0, coords0))
    pt = extract(last)
    coords = jnp.where(iota_s[:, None] == (S - 1),
                       pt[:, :, None, None], coords)
    idx_ref[...] = idxs.reshape(B, S)
    newxyz_ref[...] = coords.reshape(B, 3, S)


def _fps(xyz):
    B, _, N = xyz.shape
    return pl.pallas_call(
        _fps_body,
        out_shape=(jax.ShapeDtypeStruct((B, NPOINT), jnp.int32),
                   jax.ShapeDtypeStruct((B, 3, NPOINT), jnp.float32)),
    )(xyz.reshape(B, 3, N // _NLANE, _NLANE))


_TQ = 256


def _knn_body(newxyz_ref, xyz_ref, idx_ref):
    _, _, N = xyz_ref.shape
    q = newxyz_ref[0]
    x = xyz_ref[0]
    qt = q.T
    inner = jnp.dot(qt, x, preferred_element_type=jnp.float32)
    qsq = jnp.sum(qt * qt, axis=1, keepdims=True)
    xsq = jnp.sum(x * x, axis=0, keepdims=True)
    d = (-2.0 * inner + qsq) + xsq
    iota = lax.broadcasted_iota(jnp.int32, (_TQ, N), 1)
    cols = []
    for _ in range(NSAMPLE):
        amin = jnp.argmin(d, axis=1).astype(jnp.int32)[:, None]
        cols.append(amin)
        d = jnp.where(iota == amin, jnp.inf, d)
    idx_ref[0] = jnp.concatenate(cols, axis=1)


def _knn(new_xyz_cn, xyz):
    B, _, N = xyz.shape
    grid = (B, NPOINT // _TQ)
    return pl.pallas_call(
        _knn_body,
        grid=grid,
        in_specs=[
            pl.BlockSpec((1, 3, _TQ), lambda b, t: (b, 0, t)),
            pl.BlockSpec((1, 3, N), lambda b, t: (b, 0, 0)),
        ],
        out_specs=pl.BlockSpec((1, _TQ, NSAMPLE), lambda b, t: (b, t, 0)),
        out_shape=jax.ShapeDtypeStruct((B, NPOINT, NSAMPLE), jnp.int32),
    )(new_xyz_cn, xyz)


_GROWS = 512
_RPW = _GROWS // 32


def _sc_gather(idx2d, feats):
    C = feats.shape[1]
    mesh = plsc.VectorSubcoreMesh(core_axis_name="c", subcore_axis_name="s")

    @functools.partial(
        pl.kernel,
        mesh=mesh,
        compiler_params=pltpu.CompilerParams(use_tc_tiling_on_sc=False),
        out_type=jax.ShapeDtypeStruct((_GROWS, 128, C), jnp.float32),
        scratch_types=[
            pltpu.VMEM((_RPW, 128), jnp.int32),
            pltpu.VMEM((_RPW, 128, C), jnp.float32),
            pltpu.SemaphoreType.DMA,
        ],
    )
    def gk(idx_hbm, feats_hbm, out_hbm, idx_v, rows_v, sem):
        wid = lax.axis_index("s") * 2 + lax.axis_index("c")
        base = wid * _RPW
        pltpu.sync_copy(idx_hbm.at[pl.ds(base, _RPW)], idx_v)
        copies = [
            pltpu.async_copy(feats_hbm.at[idx_v.at[j]], rows_v.at[j], sem)
            for j in range(_RPW)
        ]
        for cp in copies:
            cp.wait()
        pltpu.sync_copy(rows_v, out_hbm.at[pl.ds(base, _RPW)])

    return gk(idx2d, feats)


_TS1 = 512


def _d1_body(nbr_ref, ctr_ref, g_ref, s_ref):
    t = pl.program_id(0)
    x = nbr_ref[...]
    c = ctr_ref[...]
    cb = jnp.broadcast_to(c[:, None, :], (_TS1, NSAMPLE, 16)).reshape(
        _TS1 * NSAMPLE, 16)
    xn = x - cb
    xb = xn.astype(jnp.bfloat16).astype(jnp.float32)
    g = lax.dot_general(xb, xb, (((0,), (0,)), ((), ())),
                        precision=lax.Precision.HIGHEST,
                        preferred_element_type=jnp.float32)
    s = jnp.broadcast_to(jnp.sum(xb, axis=0, keepdims=True), (8, 16))

    @pl.when(t == 0)
    def _():
        g_ref[...] = g
        s_ref[...] = s

    @pl.when(t > 0)
    def _():
        g_ref[...] = g_ref[...] + g
        s_ref[...] = s_ref[...] + s


def _d2_body(nbr_ref, ctr_ref, wk_ref, par_ref, agg_ref, st_ref):
    x = nbr_ref[...]
    c = ctr_ref[...]
    cb = jnp.broadcast_to(c[:, None, :], (_TS1, NSAMPLE, 16)).reshape(
        _TS1 * NSAMPLE, 16)
    xn = x - cb
    xb = xn.astype(jnp.bfloat16).astype(jnp.float32)
    wkb = wk_ref[...].astype(jnp.bfloat16).astype(jnp.float32)
    kern = lax.dot_general(xb, wkb, (((1,), (1,)), ((), ())),
                           precision=lax.Precision.HIGHEST,
                           preferred_element_type=jnp.float32)
    kern = kern * par_ref[0:1, :] + par_ref[1:2, :]
    kern = _leaky(kern).astype(jnp.bfloat16).astype(jnp.float32)
    y = jnp.sum(xb * par_ref[2:3, :], axis=1, keepdims=True)
    agg = jnp.sum((kern * y).reshape(_TS1, NSAMPLE, 16), axis=1)
    agg_ref[...] = agg
    s1 = jnp.sum(agg)
    s2 = jnp.sum(agg * agg)
    row = lax.broadcasted_iota(jnp.int32, (1, 8, 128), 1)
    st_ref[...] = jnp.where(row == 0, s1, s2)


def _d3_body(agg_ref, wl_ref, bl_ref, ab_ref, out_ref):
    a = ab_ref[0:1, 0:1]
    cshift = ab_ref[1:2, 0:1]
    y = _leaky(agg_ref[...] * a + cshift)
    yb = y.astype(jnp.bfloat16).astype(jnp.float32)
    wlb = wl_ref[...].astype(jnp.bfloat16).astype(jnp.float32)
    z = lax.dot_general(yb, wlb, (((1,), (1,)), ((), ())),
                        precision=lax.Precision.HIGHEST,
                        preferred_element_type=jnp.float32) + bl_ref[...]
    z = _leaky(z)
    out_ref[0] = z.T


def kernel(xyz, points, W_kernel, gamma_k, beta_k, W_agg, gamma_a, beta_a,
           W_lin, b_lin):
    B, _, N = xyz.shape
    S, K = NPOINT, NSAMPLE
    fps_idx, new_xyz_cn = _fps(xyz)
    idx = _knn(new_xyz_cn, xyz)

    feats = jnp.concatenate(
        [xyz.transpose(0, 2, 1), points.transpose(0, 2, 1)], axis=-1
    ).reshape(B * N, 16)
    idxf = (idx + (jnp.arange(B, dtype=jnp.int32) * N)[:, None, None]
            ).reshape(_GROWS, 128)
    nbr = _sc_gather(idxf, feats).reshape(B * S * K, 16)

    ctr16 = jnp.concatenate(
        [new_xyz_cn.transpose(0, 2, 1).reshape(B * S, 3),
         jnp.zeros((B * S, 13), jnp.float32)], axis=1)

    T1 = (B * S) // _TS1
    g, s8 = pl.pallas_call(
        _d1_body,
        grid=(T1,),
        in_specs=[
            pl.BlockSpec((_TS1 * K, 16), lambda t: (t, 0)),
            pl.BlockSpec((_TS1, 16), lambda t: (t, 0)),
        ],
        out_specs=(pl.BlockSpec((16, 16), lambda t: (0, 0)),
                   pl.BlockSpec((8, 16), lambda t: (0, 0))),
        out_shape=(jax.ShapeDtypeStruct((16, 16), jnp.float32),
                   jax.ShapeDtypeStruct((8, 16), jnp.float32)),
    )(nbr, ctr16)

    M = B * S * K
    wkb = W_kernel.astype(jnp.bfloat16).astype(jnp.float32)
    xbar = s8[0] / M
    mean = wkb @ xbar
    e2 = jnp.sum((wkb @ (g / M)) * wkb, axis=1)
    var = e2 - mean ** 2
    scale = gamma_k / jnp.sqrt(var + EPS)
    shift = beta_k - mean * scale
    params = jnp.zeros((8, 16), jnp.float32)
    waggb = W_agg[0].astype(jnp.bfloat16).astype(jnp.float32)
    params = params.at[0].set(scale).at[1].set(shift).at[2].set(waggb)

    agg, st = pl.pallas_call(
        _d2_body,
        grid=(T1,),
        in_specs=[
            pl.BlockSpec((_TS1 * K, 16), lambda t: (t, 0)),
            pl.BlockSpec((_TS1, 16), lambda t: (t, 0)),
            pl.BlockSpec((16, 16), lambda t: (0, 0)),
            pl.BlockSpec((8, 16), lambda t: (0, 0)),
        ],
        out_specs=(pl.BlockSpec((_TS1, 16), lambda t: (t, 0)),
                   pl.BlockSpec((1, 8, 128), lambda t: (t, 0, 0))),
        out_shape=(jax.ShapeDtypeStruct((B * S, 16), jnp.float32),
                   jax.ShapeDtypeStruct((T1, 8, 128), jnp.float32)),
    )(nbr, ctr16, W_kernel, params)

    M2 = B * S * 16
    s1 = jnp.sum(st[:, 0, 0])
    s2 = jnp.sum(st[:, 1, 0])
    m2 = s1 / M2
    v2 = s2 / M2 - m2 ** 2
    a = gamma_a[0] / jnp.sqrt(v2 + EPS)
    cshift = beta_a[0] - m2 * a
    ab = jnp.zeros((8, 128), jnp.float32).at[0, 0].set(a).at[1, 0].set(cshift)

    TS3 = 1024
    new_feat = pl.pallas_call(
        _d3_body,
        grid=(B, S // TS3),
        in_specs=[
            pl.BlockSpec((TS3, 16), lambda b, t: (b * (S // TS3) + t, 0)),
            pl.BlockSpec((16, 16), lambda b, t: (0, 0)),
            pl.BlockSpec((1, 16), lambda b, t: (0, 0)),
            pl.BlockSpec((8, 128), lambda b, t: (0, 0)),
        ],
        out_specs=pl.BlockSpec((1, 16, TS3), lambda b, t: (b, 0, t)),
        out_shape=jax.ShapeDtypeStruct((B, 16, S), jnp.float32),
    )(agg, W_lin, b_lin.reshape(1, 16), ab)

    return (new_xyz_cn, new_feat, fps_idx)
